# baseline (speedup 1.0000x reference)
import jax
import jax.numpy as jnp
from jax.experimental import pallas as pl
from jax.experimental.pallas import tpu as pltpu

_ROWS = 160000
_N_CHUNKS = 8
_ROWS_PER = _ROWS // _N_CHUNKS


def _copy_body(x_ref, o_ref, sems):
    copies = [
        pltpu.make_async_copy(
            x_ref.at[pl.ds(i * _ROWS_PER, _ROWS_PER), :],
            o_ref.at[pl.ds(i * _ROWS_PER, _ROWS_PER), :],
            sems.at[i],
        )
        for i in range(_N_CHUNKS)
    ]
    for c in copies:
        c.start()
    for c in copies:
        c.wait()


def kernel(x_j):
    return pl.pallas_call(
        _copy_body,
        out_shape=jax.ShapeDtypeStruct(x_j.shape, x_j.dtype),
        in_specs=[pl.BlockSpec(memory_space=pl.ANY)],
        out_specs=pl.BlockSpec(memory_space=pl.ANY),
        scratch_shapes=[pltpu.SemaphoreType.DMA((_N_CHUNKS,))],
    )(x_j)


# speedup vs baseline: 48.2356x; 48.2356x over previous
import jax
import jax.numpy as jnp
from jax.experimental import pallas as pl
from jax.experimental.pallas import tpu as pltpu

_ROWS = 160000
_COLS = 256
_BLOCK = 4000


def _copy_body(x_ref, o_ref):
    o_ref[...] = x_ref[...]


def kernel(x_j):
    grid = (_ROWS // _BLOCK,)
    return pl.pallas_call(
        _copy_body,
        grid=grid,
        in_specs=[pl.BlockSpec((_BLOCK, _COLS), lambda i: (i, 0))],
        out_specs=pl.BlockSpec((_BLOCK, _COLS), lambda i: (i, 0)),
        out_shape=jax.ShapeDtypeStruct(x_j.shape, x_j.dtype),
    )(x_j)


# speedup vs baseline: 49.1519x; 1.0190x over previous
import jax
import jax.numpy as jnp
from jax.experimental import pallas as pl
from jax.experimental.pallas import tpu as pltpu

_ROWS = 160000
_COLS = 256
_BLOCK = 8000


def _copy_body(x_ref, o_ref):
    o_ref[...] = x_ref[...]


def kernel(x_j):
    grid = (_ROWS // _BLOCK,)
    return pl.pallas_call(
        _copy_body,
        grid=grid,
        in_specs=[pl.BlockSpec((_BLOCK, _COLS), lambda i: (i, 0))],
        out_specs=pl.BlockSpec((_BLOCK, _COLS), lambda i: (i, 0)),
        out_shape=jax.ShapeDtypeStruct(x_j.shape, x_j.dtype),
        compiler_params=pltpu.CompilerParams(
            dimension_semantics=("parallel",),
        ),
    )(x_j)
